# baseline (speedup 1.0000x reference)
import dataclasses
import functools

import numpy as np

import jax
import jax.numpy as jnp
from jax import lax
from jax.experimental import pallas as pl
from jax.experimental.pallas import tpu as pltpu
from jax.experimental.pallas import tpu_sc as plsc

_W = 128


def _prep_body(lpt_ref, p0t_ref):
    p0t_ref[...] = jnp.exp(lpt_ref[...]).astype(jnp.bfloat16)


def _sc_selected_sum(tab128, ridx, lane):
    t1 = ridx.shape[0]
    mesh = plsc.VectorSubcoreMesh(core_axis_name="c", subcore_axis_name="s")
    cp = pltpu.CompilerParams()
    if "needs_layout_passes" in pltpu.CompilerParams.__dataclass_fields__:
        cp = dataclasses.replace(cp, needs_layout_passes=False)

    @functools.partial(
        pl.kernel,
        out_type=jax.ShapeDtypeStruct(
            (mesh.num_cores, mesh.num_subcores, 16), jnp.float32),
        mesh=mesh,
        compiler_params=cp,
        scratch_types=[
            pltpu.VMEM((_W, 128), jnp.float32),
            pltpu.VMEM((16,), jnp.float32),
        ],
    )
    def sel_kernel(tab_hbm, ridx_hbm, lane_hbm, out_hbm, g_ref, acc_ref):
        c = lax.axis_index("c")
        s = lax.axis_index("s")
        acc_ref[...] = jnp.zeros((16,), jnp.float32)

        def body(ridx_vmem, lane_vmem):
            pltpu.sync_copy(tab_hbm.at[ridx_vmem.at[0]], g_ref)

            @pl.loop(0, _W, step=16)
            def _(i):
                rows = lax.iota(jnp.int32, 16) + i
                lanes = lane_vmem[0, pl.ds(i, 16)]
                acc_ref[...] += plsc.load_gather(g_ref, [rows, lanes])

        pltpu.emit_pipeline(
            body,
            grid=(t1 // _W,),
            in_specs=[
                pl.BlockSpec((1, _W), index_map=lambda i: (0, i)),
                pl.BlockSpec((1, _W), index_map=lambda i: (0, i)),
            ],
            out_specs=[],
            core_axis_name=("c", "s"),
            dimension_semantics=(pltpu.PARALLEL,),
        )(ridx_hbm, lane_hbm)

        pltpu.sync_copy(acc_ref, out_hbm.at[c, s])

    return sel_kernel(tab128, ridx.reshape(1, t1), lane.reshape(1, t1))


def _half(u, xc, xn, wst, p0t):
    corr = jnp.dot(u.astype(jnp.bfloat16), wst,
                   preferred_element_type=jnp.float32)
    lane = lax.broadcasted_iota(jnp.int32, corr.shape, 1)
    m = jnp.max(corr, axis=1, keepdims=True)
    e = jnp.exp((corr - m).astype(jnp.bfloat16))
    mz = jnp.dot(e, p0t, preferred_element_type=jnp.float32)
    z = jnp.sum(jnp.where(lane == xc, mz, 0.0), axis=1)
    selc = jnp.sum(jnp.where(lane == xn, corr, 0.0), axis=1)
    return jnp.sum(m[:, 0] + jnp.log(z) - selc)


def _main_body(base_ref, u_ref, xc_ref, xn_ref, wst_ref, p0t_ref, out_ref):
    i = pl.program_id(0)
    h = u_ref.shape[0] // 2
    wst = wst_ref[...]
    p0t = p0t_ref[...]
    b0 = _half(u_ref[:h], xc_ref[:h], xn_ref[:h], wst, p0t)
    b1 = _half(u_ref[h:], xc_ref[h:], xn_ref[h:], wst, p0t)
    block = b0 + b1

    @pl.when(i == 0)
    def _():
        out_ref[...] = jnp.zeros((1, 1), jnp.float32)

    out_ref[...] += block.reshape(1, 1)


def _device_part(x_seq, u_seq, wst, lpt, tab128, ndev, bt, u_dim, npad):
    t1 = (x_seq.shape[0] - 1) // ndev
    b = bt
    while t1 % b != 0:
        b //= 2

    idx = lax.axis_index("d")
    xs = lax.dynamic_slice_in_dim(x_seq, idx * t1, t1 + 1).astype(jnp.int32)
    xc_sh = xs[:-1]
    xn_sh = xs[1:]
    flat = xc_sh * npad + xn_sh
    ridx_sh = flat // 128
    lane_sh = flat % 128

    sc_partials = _sc_selected_sum(tab128, ridx_sh, lane_sh)

    p0t = pl.pallas_call(
        _prep_body,
        out_shape=jax.ShapeDtypeStruct((npad, npad), jnp.bfloat16),
    )(lpt)

    base = (idx * (t1 // b)).astype(jnp.int32).reshape(1)
    out = pl.pallas_call(
        _main_body,
        grid_spec=pltpu.PrefetchScalarGridSpec(
            num_scalar_prefetch=1,
            grid=(t1 // b,),
            in_specs=[
                pl.BlockSpec((b, u_dim), lambda i, base: (base[0] + i, 0)),
                pl.BlockSpec((b, 1), lambda i, base: (i, 0)),
                pl.BlockSpec((b, 1), lambda i, base: (i, 0)),
                pl.BlockSpec((u_dim, npad), lambda i, base: (0, 0)),
                pl.BlockSpec((npad, npad), lambda i, base: (0, 0)),
            ],
            out_specs=pl.BlockSpec((1, 1), lambda i, base: (0, 0)),
        ),
        out_shape=jax.ShapeDtypeStruct((1, 1), jnp.float32),
        compiler_params=pltpu.CompilerParams(
            dimension_semantics=("arbitrary",),
        ),
    )(base, u_seq, xc_sh.reshape(t1, 1), xn_sh.reshape(t1, 1), wst, p0t)

    return out[0, 0] - jnp.sum(sc_partials)


def kernel(x_seq, u_seq, logP0, ws):
    n = logP0.shape[0]
    u_dim = u_seq.shape[1]
    t1 = x_seq.shape[0] - 1
    npad = ((n + 127) // 128) * 128
    bt = 1024

    wst = jnp.pad(ws, ((0, npad - n), (0, 0))).T.astype(jnp.bfloat16)
    lp_pad = jnp.pad(logP0, ((0, npad - n), (0, npad - n)),
                     constant_values=-1e30)
    tab128 = lp_pad.reshape(npad * npad // 128, 128)

    ndev = jax.device_count()
    while ndev > 1 and t1 % (ndev * bt) != 0:
        ndev -= 1
    mesh = jax.sharding.Mesh(np.array(jax.devices()[:ndev]), ("d",))
    part = functools.partial(_device_part, ndev=ndev, bt=bt, u_dim=u_dim,
                             npad=npad)
    p = jax.sharding.PartitionSpec
    run = jax.shard_map(
        lambda *a: lax.psum(part(*a), "d"),
        mesh=mesh,
        in_specs=(p(), p(), p(), p(), p()),
        out_specs=p(),
        check_vma=False,
    )
    with jax.sharding.use_abstract_mesh(mesh.abstract_mesh):
        return run(x_seq, u_seq, wst, lp_pad.T, tab128)


# speedup vs baseline: 1.1290x; 1.1290x over previous
import dataclasses
import functools

import numpy as np

import jax
import jax.numpy as jnp
from jax import lax
from jax.experimental import pallas as pl
from jax.experimental.pallas import tpu as pltpu
from jax.experimental.pallas import tpu_sc as plsc

_W = 128


def _prep_body(lpt_ref, p0t_ref):
    p0t_ref[...] = jnp.exp(lpt_ref[...]).astype(jnp.bfloat16)


def _sc_selected_sums(tab128, wsp, u_hbm, ridx, lane, xn):
    t1 = ridx.shape[0]
    u_dim = u_hbm.shape[1]
    mesh = plsc.VectorSubcoreMesh(core_axis_name="c", subcore_axis_name="s")
    cp = pltpu.CompilerParams()
    if "needs_layout_passes" in pltpu.CompilerParams.__dataclass_fields__:
        cp = dataclasses.replace(cp, needs_layout_passes=False)

    @functools.partial(
        pl.kernel,
        out_type=jax.ShapeDtypeStruct(
            (mesh.num_cores, mesh.num_subcores, 16), jnp.float32),
        mesh=mesh,
        compiler_params=cp,
        scratch_types=[
            pltpu.VMEM((_W, 128), jnp.float32),
            pltpu.VMEM((_W, 128), jnp.float32),
            pltpu.VMEM((16,), jnp.float32),
        ],
    )
    def sel_kernel(tab_hbm, wsp_hbm, u_ref, ridx_hbm, lane_hbm, xn_hbm,
                   out_hbm, g_ref, wg_ref, acc_ref):
        c = lax.axis_index("c")
        s = lax.axis_index("s")
        acc_ref[...] = jnp.zeros((16,), jnp.float32)

        def body(ridx_vmem, lane_vmem, xn_vmem, u_vmem):
            pltpu.sync_copy(tab_hbm.at[ridx_vmem.at[0]], g_ref)
            pltpu.sync_copy(wsp_hbm.at[xn_vmem.at[0]], wg_ref)

            @pl.loop(0, _W, step=16)
            def _(i):
                rows = lax.iota(jnp.int32, 16) + i
                lanes = lane_vmem[0, pl.ds(i, 16)]
                acc_ref[...] += plsc.load_gather(g_ref, [rows, lanes])

            @pl.loop(0, _W)
            def _(t):
                for k in range(u_dim // 16):
                    acc_ref[...] += (u_vmem[t, pl.ds(16 * k, 16)]
                                     * wg_ref[t, pl.ds(16 * k, 16)])

        pltpu.emit_pipeline(
            body,
            grid=(t1 // _W,),
            in_specs=[
                pl.BlockSpec((1, _W), index_map=lambda i: (0, i)),
                pl.BlockSpec((1, _W), index_map=lambda i: (0, i)),
                pl.BlockSpec((1, _W), index_map=lambda i: (0, i)),
                pl.BlockSpec((_W, u_dim), index_map=lambda i: (i, 0)),
            ],
            out_specs=[],
            core_axis_name=("c", "s"),
            dimension_semantics=(pltpu.PARALLEL,),
        )(ridx_hbm, lane_hbm, xn_hbm, u_ref)

        pltpu.sync_copy(acc_ref, out_hbm.at[c, s])

    return sel_kernel(tab128, wsp, u_hbm, ridx.reshape(1, t1),
                      lane.reshape(1, t1), xn.reshape(1, t1))


def _half(u, xc, wst, p0t):
    corr = jnp.dot(u.astype(jnp.bfloat16), wst,
                   preferred_element_type=jnp.float32)
    lane = lax.broadcasted_iota(jnp.int32, corr.shape, 1)
    m = jnp.max(corr, axis=1, keepdims=True)
    e = jnp.exp((corr - m).astype(jnp.bfloat16))
    mz = jnp.dot(e, p0t, preferred_element_type=jnp.float32)
    z = jnp.sum(jnp.where(lane == xc, mz, 0.0), axis=1)
    return jnp.sum(m[:, 0] + jnp.log(z))


def _main_body(u_ref, xc_ref, wst_ref, p0t_ref, out_ref):
    i = pl.program_id(0)
    h = u_ref.shape[0] // 2
    wst = wst_ref[...]
    p0t = p0t_ref[...]
    b0 = _half(u_ref[:h], xc_ref[:h], wst, p0t)
    b1 = _half(u_ref[h:], xc_ref[h:], wst, p0t)
    block = b0 + b1

    @pl.when(i == 0)
    def _():
        out_ref[...] = jnp.zeros((1, 1), jnp.float32)

    out_ref[...] += block.reshape(1, 1)


def _device_part(x_seq, u_seq, wst, lpt, tab128, wsp, ndev, bt, u_dim, npad):
    t1 = (x_seq.shape[0] - 1) // ndev
    b = bt
    while t1 % b != 0:
        b //= 2

    idx = lax.axis_index("d")
    xs = lax.dynamic_slice_in_dim(x_seq, idx * t1, t1 + 1).astype(jnp.int32)
    u_sh = lax.dynamic_slice_in_dim(u_seq, idx * t1, t1)
    xc_sh = xs[:-1]
    xn_sh = xs[1:]
    flat = xc_sh * npad + xn_sh
    ridx_sh = flat // 128
    lane_sh = flat % 128

    sc_partials = _sc_selected_sums(tab128, wsp, u_sh, ridx_sh, lane_sh, xn_sh)

    p0t = pl.pallas_call(
        _prep_body,
        out_shape=jax.ShapeDtypeStruct((npad, npad), jnp.bfloat16),
    )(lpt)

    out = pl.pallas_call(
        _main_body,
        grid=(t1 // b,),
        in_specs=[
            pl.BlockSpec((b, u_dim), lambda i: (i, 0)),
            pl.BlockSpec((b, 1), lambda i: (i, 0)),
            pl.BlockSpec((u_dim, npad), lambda i: (0, 0)),
            pl.BlockSpec((npad, npad), lambda i: (0, 0)),
        ],
        out_specs=pl.BlockSpec((1, 1), lambda i: (0, 0)),
        out_shape=jax.ShapeDtypeStruct((1, 1), jnp.float32),
        compiler_params=pltpu.CompilerParams(
            dimension_semantics=("arbitrary",),
        ),
    )(u_sh, xc_sh.reshape(t1, 1), wst, p0t)

    return out[0, 0] - jnp.sum(sc_partials)


def kernel(x_seq, u_seq, logP0, ws):
    n = logP0.shape[0]
    u_dim = u_seq.shape[1]
    t1 = x_seq.shape[0] - 1
    npad = ((n + 127) // 128) * 128
    bt = 1024

    wst = jnp.pad(ws, ((0, npad - n), (0, 0))).T.astype(jnp.bfloat16)
    wsp = jnp.pad(ws, ((0, npad - n), (0, 128 - u_dim)))
    lp_pad = jnp.pad(logP0, ((0, npad - n), (0, npad - n)),
                     constant_values=-1e30)
    tab128 = lp_pad.reshape(npad * npad // 128, 128)

    ndev = jax.device_count()
    while ndev > 1 and t1 % (ndev * bt) != 0:
        ndev -= 1
    mesh = jax.sharding.Mesh(np.array(jax.devices()[:ndev]), ("d",))
    part = functools.partial(_device_part, ndev=ndev, bt=bt, u_dim=u_dim,
                             npad=npad)
    p = jax.sharding.PartitionSpec
    run = jax.shard_map(
        lambda *a: lax.psum(part(*a), "d"),
        mesh=mesh,
        in_specs=(p(), p(), p(), p(), p(), p()),
        out_specs=p(),
        check_vma=False,
    )
    with jax.sharding.use_abstract_mesh(mesh.abstract_mesh):
        return run(x_seq, u_seq, wst, lp_pad.T, tab128, wsp)
